# baseline (speedup 1.0000x reference)
import functools

import jax
import jax.numpy as jnp
from jax import lax
from jax.experimental import pallas as pl
from jax.experimental.pallas import tpu as pltpu
from jax.experimental.pallas import tpu_sc as plsc

BATCH = 16384
XCOLS = 130
CF_DIM = 16
NROW = 1000000
BLK = 8
NBLK = NROW // BLK
NC = 2
NS = 16
NW = NC * NS
BPW = BATCH // NW
CPC = 32
NCH = BPW // CPC
GPC = CPC // 16


def _rec_body(x_hbm, ws_hbm, nnwt_hbm, icwt_hbm, ucwt_hbm,
              cfu_hbm, cfi_hbm, nnu_hbm, nni_hbm,
              out_hbm,
              x_v, ws_v, nnwt_v, icwt_v, ucwt_v,
              ublk_v, iblk_v, usub_v, isub_v,
              cfu_a, cfi_a, nnu_a, nni_a,
              cfu_b, cfi_b, nnu_b, nni_b,
              out_v,
              sem_x, sem_a, sem_b):
    wid = lax.axis_index("s") * NC + lax.axis_index("c")
    base = wid * BPW

    cp_x = pltpu.async_copy(x_hbm.at[pl.ds(base, BPW)], x_v, sem_x)
    pltpu.sync_copy(ws_hbm, ws_v)
    pltpu.sync_copy(nnwt_hbm, nnwt_v)
    pltpu.sync_copy(icwt_hbm, icwt_v)
    pltpu.sync_copy(ucwt_hbm, ucwt_v)
    cp_x.wait()

    lanes = lax.iota(jnp.int32, 16)
    zeros16 = jnp.zeros((16,), jnp.int32)
    ones16 = jnp.full((16,), 1, jnp.int32)

    def extract(g, _):
        rows = g * 16 + lanes
        u = plsc.load_gather(x_v, [rows, zeros16])
        i = plsc.load_gather(x_v, [rows, ones16])
        ch = g // GPC
        off = (g % GPC) * 16
        ublk_v[ch, pl.ds(off, 16)] = lax.shift_right_logical(u, 3)
        iblk_v[ch, pl.ds(off, 16)] = lax.shift_right_logical(i, 3)
        usub_v[pl.ds(g * 16, 16)] = (u & 7) * 16
        isub_v[pl.ds(g * 16, 16)] = (i & 7) * 16
        return 0

    lax.fori_loop(0, BPW // 16, extract, 0)

    def fire(ch, bufs, sem):
        pltpu.async_copy(cfu_hbm.at[ublk_v.at[ch]], bufs[0], sem)
        pltpu.async_copy(cfi_hbm.at[iblk_v.at[ch]], bufs[1], sem)
        pltpu.async_copy(nnu_hbm.at[ublk_v.at[ch]], bufs[2], sem)
        pltpu.async_copy(nni_hbm.at[iblk_v.at[ch]], bufs[3], sem)

    def drain(ch, bufs, sem):
        pltpu.make_async_copy(cfu_hbm.at[ublk_v.at[ch]], bufs[0], sem).wait()
        pltpu.make_async_copy(cfi_hbm.at[iblk_v.at[ch]], bufs[1], sem).wait()
        pltpu.make_async_copy(nnu_hbm.at[ublk_v.at[ch]], bufs[2], sem).wait()
        pltpu.make_async_copy(nni_hbm.at[iblk_v.at[ch]], bufs[3], sem).wait()

    bufs_a = (cfu_a, cfi_a, nnu_a, nni_a)
    bufs_b = (cfu_b, cfi_b, nnu_b, nni_b)

    fire(0, bufs_a, sem_a)
    fire(1, bufs_b, sem_b)

    w_nn = ws_v[0, :]
    w_ic = ws_v[1, :]
    w_uc = ws_v[2, :]
    scal = ws_v[6, :]
    a_u = jnp.zeros((16,), jnp.float32)
    a_i = jnp.zeros((16,), jnp.float32)
    for k in range(16):
        wk = w_nn[k]
        a_u = a_u + nnwt_v[k, pl.ds(0, 16)] * wk
        a_i = a_i + nnwt_v[k, pl.ds(16, 16)] * wk
    p = []
    q = []
    for c in range(4):
        pc = jnp.zeros((16,), jnp.float32)
        qc = jnp.zeros((16,), jnp.float32)
        for k in range(16):
            pc = pc + icwt_v[k, pl.ds(c * 16, 16)] * w_ic[k]
            qc = qc + ucwt_v[k, pl.ds(c * 16, 16)] * w_uc[k]
        p.append(pc)
        q.append(qc)
    w_cf = scal[0]
    const = (scal[1]
             + jnp.sum(ws_v[3, :] * w_nn)
             + jnp.sum(ws_v[4, :] * w_ic)
             + jnp.sum(ws_v[5, :] * w_uc))
    a_u_s = [a_u[c] for c in range(16)]
    a_i_s = [a_i[c] for c in range(16)]
    p_s = [p[c // 16][c % 16] for c in range(64)]
    q_s = [q[c // 16][c % 16] for c in range(64)]

    def compute(ch, bufs):
        cfu_c, cfi_c, nnu_c, nni_c = bufs
        for gg in range(GPC):
            gbase = ch * CPC + gg * 16
            rows = gg * 16 + lanes
            grow = gbase + lanes
            usub = usub_v[pl.ds(gbase, 16)]
            isub = isub_v[pl.ds(gbase, 16)]
            acc = jnp.zeros((16,), jnp.float32)
            for c in range(16):
                gu = plsc.load_gather(cfu_c, [rows, usub + c])
                gi = plsc.load_gather(cfi_c, [rows, isub + c])
                acc = acc + gu * gi
            acc = acc * w_cf
            for c in range(16):
                acc = acc + plsc.load_gather(nnu_c, [rows, usub + c]) * a_u_s[c]
                acc = acc + plsc.load_gather(nni_c, [rows, isub + c]) * a_i_s[c]
            for c in range(64):
                xc = plsc.load_gather(x_v, [grow, zeros16 + (2 + c)])
                acc = acc + xc.astype(jnp.float32) * p_s[c]
            for c in range(64):
                xc = plsc.load_gather(x_v, [grow, zeros16 + (66 + c)])
                acc = acc + xc.astype(jnp.float32) * q_s[c]
            out_v[pl.ds(gbase, 16)] = acc + const

    def body(k, _):
        ch = k * 2
        drain(ch, bufs_a, sem_a)
        compute(ch, bufs_a)

        @pl.when(k < NCH // 2 - 1)
        def _():
            fire(ch + 2, bufs_a, sem_a)

        drain(ch + 1, bufs_b, sem_b)
        compute(ch + 1, bufs_b)

        @pl.when(k < NCH // 2 - 1)
        def _():
            fire(ch + 3, bufs_b, sem_b)

        return 0

    lax.fori_loop(0, NCH // 2, body, 0)

    pltpu.sync_copy(out_v, out_hbm.at[pl.ds(base, BPW)])


@jax.jit
def _rec_call(x, ws, nnwt, icwt, ucwt, cfu, cfi, nnu, nni):
    cfu = cfu.reshape(NBLK, BLK * CF_DIM)
    cfi = cfi.reshape(NBLK, BLK * CF_DIM)
    nnu = nnu.reshape(NBLK, BLK * CF_DIM)
    nni = nni.reshape(NBLK, BLK * CF_DIM)
    mesh = plsc.VectorSubcoreMesh(core_axis_name="c", subcore_axis_name="s")
    gbuf = pltpu.VMEM((CPC, BLK * CF_DIM), jnp.float32)
    f = functools.partial(
        pl.kernel,
        out_type=jax.ShapeDtypeStruct((BATCH,), jnp.float32),
        mesh=mesh,
        compiler_params=pltpu.CompilerParams(
            use_tc_tiling_on_sc=False, needs_layout_passes=False),
        scratch_types=[
            pltpu.VMEM((BPW, XCOLS), jnp.int32),
            pltpu.VMEM((8, 16), jnp.float32),
            pltpu.VMEM((16, 32), jnp.float32),
            pltpu.VMEM((16, 64), jnp.float32),
            pltpu.VMEM((16, 64), jnp.float32),
            pltpu.VMEM((NCH, CPC), jnp.int32),
            pltpu.VMEM((NCH, CPC), jnp.int32),
            pltpu.VMEM((BPW,), jnp.int32),
            pltpu.VMEM((BPW,), jnp.int32),
            gbuf, gbuf, gbuf, gbuf,
            gbuf, gbuf, gbuf, gbuf,
            pltpu.VMEM((BPW,), jnp.float32),
            pltpu.SemaphoreType.DMA,
            pltpu.SemaphoreType.DMA,
            pltpu.SemaphoreType.DMA,
        ],
    )(_rec_body)
    return f(x, ws, nnwt, icwt, ucwt, cfu, cfi, nnu, nni)


def kernel(x, item_context_features_len, cf_user_emb, cf_item_emb,
           nn_user_emb, nn_item_emb, nn_fc_w, nn_fc_b, ic_w, ic_b,
           uc_w, uc_b, fc_w, fc_b):
    fcv = fc_w[:, 0]
    scal = jnp.zeros((16,), jnp.float32).at[0].set(fcv[0]).at[1].set(fc_b[0])
    ws = jnp.stack([fcv[1:17], fcv[17:33], fcv[33:49],
                    nn_fc_b, ic_b, uc_b, scal,
                    jnp.zeros((16,), jnp.float32)])
    return _rec_call(x, ws, nn_fc_w.T, ic_w.T, uc_w.T,
                     cf_user_emb, cf_item_emb, nn_user_emb, nn_item_emb)
